# baseline (speedup 1.0000x reference)
import functools
import math

import jax
import jax.numpy as jnp
import numpy as np
from jax import lax
from jax.experimental import pallas as pl
from jax.experimental.pallas import tpu as pltpu
from jax.experimental.pallas import tpu_sc as plsc

NLAT = 128
NLON = 256
LMAX = 50
MMAX = 50
N = 1024
B = 2
G = NLAT * NLON
TOTQ = B * G
NWORKER = 32
QW = TOTQ // NWORKER
QVECS = QW // 16

_PI = math.pi


def _cc_quad_weights(n):
    tj = np.pi * np.arange(n) / (n - 1)
    x = np.cos(tj)
    Nn = n - 1
    w = np.zeros(n)
    for j in range(n):
        tmp = 0.0
        for k in range(1, Nn // 2 + 1):
            bk = 1.0 if 2 * k == Nn else 2.0
            tmp += bk / (4.0 * k * k - 1.0) * np.cos(2.0 * k * tj[j])
        wj = 1.0 - tmp
        wj = wj / Nn if (j == 0 or j == Nn) else 2.0 * wj / Nn
        w[j] = wj
    return x, w


def _legpoly(mmax, lmax, x):
    nlat = x.shape[0]
    pct = np.zeros((mmax, lmax, nlat))
    sint = np.sqrt(np.clip(1.0 - x * x, 0.0, None))
    pmm = np.full(nlat, math.sqrt(1.0 / (4.0 * math.pi)))
    for m in range(mmax):
        if m > 0:
            pmm = -math.sqrt((2.0 * m + 1.0) / (2.0 * m)) * sint * pmm
        if m < lmax:
            pct[m, m] = pmm
        if m + 1 < lmax:
            pct[m, m + 1] = math.sqrt(2.0 * m + 3.0) * x * pmm
        for l in range(m + 2, lmax):
            a = math.sqrt((4.0 * l * l - 1.0) / (l * l - m * m))
            b = math.sqrt((((l - 1.0) ** 2) - m * m) / (4.0 * (l - 1.0) ** 2 - 1.0))
            pct[m, l] = a * (x * pct[m, l - 1] - b * pct[m, l - 2])
    return pct


_COST, _WQ = _cc_quad_weights(NLAT)
_SHT_W = (_legpoly(MMAX, LMAX, _COST) * _WQ[None, None, :]).astype(np.float32)
_WT = np.ascontiguousarray(np.transpose(_SHT_W, (2, 1, 0)))
_j = np.arange(NLON)[:, None].astype(np.float64)
_m = np.arange(MMAX)[None, :].astype(np.float64)
_COS = np.cos(2.0 * np.pi * _j * _m / NLON).astype(np.float32)
_RW = np.exp(-((LMAX - np.arange(1, LMAX + 1)) ** 2) / (2.0 * LMAX ** 2)).astype(np.float32)[:, None]


def _sph_kernel(px_ref, py_ref, pz_ref, sxsp_ref, sysp_ref, ftsp_ref):
    x = px_ref[0]
    y = py_ref[0]
    z = pz_ref[0]
    r = jnp.sqrt(x * x + y * y + z * z)
    rho = jnp.sqrt(y * y + z * z)

    def acos(v):
        return jnp.arctan2(jnp.sqrt((1.0 - v) * (1.0 + v)), v)

    theta = acos(jnp.clip(x / r, -1.0, 1.0))
    a = acos(jnp.clip(y / rho, -1.0, 1.0))
    phi = jnp.where(z < 0.0, 2.0 * _PI - a, a) - _PI
    sxsp_ref[...] = jnp.broadcast_to(theta.reshape(N, 1), (N, 16)).reshape(1, N, 16)
    sysp_ref[...] = jnp.broadcast_to(phi.reshape(N, 1), (N, 16)).reshape(1, N, 16)
    ftsp_ref[...] = jnp.broadcast_to(r.reshape(N, 1), (N, 16)).reshape(1, N, 16)


_SC_MESH = plsc.VectorSubcoreMesh(core_axis_name="c", subcore_axis_name="s")


@functools.partial(
    pl.kernel,
    mesh=_SC_MESH,
    out_type=[jax.ShapeDtypeStruct((TOTQ,), jnp.float32) for _ in range(6)],
    scratch_types=[
        pltpu.VMEM((N * 16,), jnp.float32),
        pltpu.VMEM((N * 16,), jnp.float32),
        pltpu.VMEM((N * 16,), jnp.float32),
        pltpu.VMEM((QW // 2,), jnp.float32),
        pltpu.VMEM((QW // 2,), jnp.float32),
        pltpu.VMEM((QW // 2,), jnp.float32),
        pltpu.VMEM((QW // 2,), jnp.float32),
        pltpu.VMEM((QW // 2,), jnp.float32),
        pltpu.VMEM((QW // 2,), jnp.float32),
    ],
)
def _sc_knn(sxsp_hbm, sysp_hbm, ftsp_hbm,
            o_f1, o_f2, o_f3, o_d1, o_d2, o_d3,
            sxspv, syspv, ftspv,
            bf1, bf2, bf3, bd1, bd2, bd3):
    wid = lax.axis_index("c") * 16 + lax.axis_index("s")
    base = wid * QW
    b = base >> 15

    pltpu.sync_copy(sxsp_hbm.at[b], sxspv)
    pltpu.sync_copy(sysp_hbm.at[b], syspv)
    pltpu.sync_copy(ftsp_hbm.at[b], ftspv)

    lanes = lax.iota(jnp.int32, 16)
    scale = jnp.float32(1.0 / NLAT)
    pif = jnp.float32(_PI)
    inf = jnp.full((16,), jnp.inf, jnp.float32)
    zero = jnp.zeros((16,), jnp.float32)
    half = QW // 2

    for h in range(2):
        def qstep(i, carry):
            flat = base + h * half + i * 16
            fv = lanes + flat
            row = lax.shift_right_logical(fv, 8) & 127
            col = fv & 255
            tq = (row.astype(jnp.float32) * scale) * pif
            pq = ((col.astype(jnp.float32) - np.float32(NLAT)) * scale) * pif

            def chunk(c, ks):
                m1, m2, m3, f1, f2, f3 = ks
                for j in range(16):
                    rx = sxspv[pl.ds(c * 256 + j * 16, 16)]
                    ry = syspv[pl.ds(c * 256 + j * 16, 16)]
                    rf = ftspv[pl.ds(c * 256 + j * 16, 16)]
                    dx = tq - rx
                    dy = pq - ry
                    d2 = dx * dx + dy * dy
                    c1 = d2 < m1
                    c2 = d2 < m2
                    c3 = d2 < m3
                    m3 = jnp.where(c2, m2, jnp.where(c3, d2, m3))
                    f3 = jnp.where(c2, f2, jnp.where(c3, rf, f3))
                    m2 = jnp.where(c1, m1, jnp.where(c2, d2, m2))
                    f2 = jnp.where(c1, f1, jnp.where(c2, rf, f2))
                    m1 = jnp.where(c1, d2, m1)
                    f1 = jnp.where(c1, rf, f1)
                return m1, m2, m3, f1, f2, f3

            m1, m2, m3, f1, f2, f3 = lax.fori_loop(
                0, N // 16, chunk, (inf, inf, inf, zero, zero, zero))

            off = pl.ds(i * 16, 16)
            bd1[off] = m1
            bd2[off] = m2
            bd3[off] = m3
            bf1[off] = f1
            bf2[off] = f2
            bf3[off] = f3
            return carry

        lax.fori_loop(0, half // 16, qstep, 0)

        dst = pl.ds(base + h * half, half)
        pltpu.sync_copy(bf1, o_f1.at[dst])
        pltpu.sync_copy(bf2, o_f2.at[dst])
        pltpu.sync_copy(bf3, o_f3.at[dst])
        pltpu.sync_copy(bd1, o_d1.at[dst])
        pltpu.sync_copy(bd2, o_d2.at[dst])
        pltpu.sync_copy(bd3, o_d3.at[dst])


def _sht_kernel(f1_ref, f2_ref, f3_ref, d1_ref, d2_ref, d3_ref,
                t_ref, cos_ref, wt_ref, rw_ref, out_ref):
    w1 = jnp.sqrt(jnp.maximum(d1_ref[...], 1e-12))
    w2 = jnp.sqrt(jnp.maximum(d2_ref[...], 1e-12))
    w3 = jnp.sqrt(jnp.maximum(d3_ref[...], 1e-12))
    x = (f1_ref[...] * w1 + f2_ref[...] * w2 + f3_ref[...] * w3) / (w1 + w2 + w3)
    xr = lax.dot(x, cos_ref[...], precision=lax.Precision.HIGHEST,
                 preferred_element_type=jnp.float32)
    xr = xr * np.float32(2.0 * _PI / NLON)
    wt = wt_ref[...]
    rw = rw_ref[...]
    loss = jnp.float32(0.0)
    for b in range(B):
        xb = xr[b * NLAT:(b + 1) * NLAT]
        cb = jnp.sum(wt * xb[:, None, :], axis=0)
        resid = cb - t_ref[b]
        loss = loss + jnp.sum(resid * resid * rw)
    out_ref[...] = (loss / B).reshape(1, 1)


def kernel(pred, target_coeffs):
    px = pred[:, :, 0].reshape(B, 1, N)
    py = pred[:, :, 1].reshape(B, 1, N)
    pz = pred[:, :, 2].reshape(B, 1, N)

    sxsp, sysp, ftsp = pl.pallas_call(
        _sph_kernel,
        grid=(B,),
        in_specs=[pl.BlockSpec((1, 1, N), lambda b: (b, 0, 0))] * 3,
        out_specs=[pl.BlockSpec((1, N, 16), lambda b: (b, 0, 0))] * 3,
        out_shape=[jax.ShapeDtypeStruct((B, N, 16), jnp.float32)] * 3,
    )(px, py, pz)

    f1, f2, f3, d1, d2, d3 = _sc_knn(
        sxsp.reshape(B, N * 16), sysp.reshape(B, N * 16), ftsp.reshape(B, N * 16))

    shp = (B * NLAT, NLON)
    loss = pl.pallas_call(
        _sht_kernel,
        out_shape=jax.ShapeDtypeStruct((1, 1), jnp.float32),
    )(f1.reshape(shp), f2.reshape(shp), f3.reshape(shp),
      d1.reshape(shp), d2.reshape(shp), d3.reshape(shp),
      target_coeffs, jnp.asarray(_COS), jnp.asarray(_WT), jnp.asarray(_RW))
    return loss[0, 0]


# speedup vs baseline: 1.3166x; 1.3166x over previous
import functools
import math

import jax
import jax.numpy as jnp
import numpy as np
from jax import lax
from jax.experimental import pallas as pl
from jax.experimental.pallas import tpu as pltpu
from jax.experimental.pallas import tpu_sc as plsc

NLAT = 128
NLON = 256
LMAX = 50
MMAX = 50
N = 1024
B = 2
G = NLAT * NLON
TOTQ = B * G
QW = 2048
NCHUNK = N // 16

_PI = math.pi


def _cc_quad_weights(n):
    tj = np.pi * np.arange(n) / (n - 1)
    x = np.cos(tj)
    Nn = n - 1
    w = np.zeros(n)
    for j in range(n):
        tmp = 0.0
        for k in range(1, Nn // 2 + 1):
            bk = 1.0 if 2 * k == Nn else 2.0
            tmp += bk / (4.0 * k * k - 1.0) * np.cos(2.0 * k * tj[j])
        wj = 1.0 - tmp
        wj = wj / Nn if (j == 0 or j == Nn) else 2.0 * wj / Nn
        w[j] = wj
    return x, w


def _legpoly(mmax, lmax, x):
    nlat = x.shape[0]
    pct = np.zeros((mmax, lmax, nlat))
    sint = np.sqrt(np.clip(1.0 - x * x, 0.0, None))
    pmm = np.full(nlat, math.sqrt(1.0 / (4.0 * math.pi)))
    for m in range(mmax):
        if m > 0:
            pmm = -math.sqrt((2.0 * m + 1.0) / (2.0 * m)) * sint * pmm
        if m < lmax:
            pct[m, m] = pmm
        if m + 1 < lmax:
            pct[m, m + 1] = math.sqrt(2.0 * m + 3.0) * x * pmm
        for l in range(m + 2, lmax):
            a = math.sqrt((4.0 * l * l - 1.0) / (l * l - m * m))
            b = math.sqrt((((l - 1.0) ** 2) - m * m) / (4.0 * (l - 1.0) ** 2 - 1.0))
            pct[m, l] = a * (x * pct[m, l - 1] - b * pct[m, l - 2])
    return pct


_COST, _WQ = _cc_quad_weights(NLAT)
_SHT_W = (_legpoly(MMAX, LMAX, _COST) * _WQ[None, None, :]).astype(np.float32)
_WT = np.ascontiguousarray(np.transpose(_SHT_W, (2, 1, 0)))
_ROWPERM = np.array([(q // 8) + 16 * (q % 8) for q in range(NLAT)])
_WTP = np.ascontiguousarray(_WT[_ROWPERM])
_j = np.arange(NLON)[:, None].astype(np.float64)
_m = np.arange(MMAX)[None, :].astype(np.float64)
_COS = np.cos(2.0 * np.pi * _j * _m / NLON).astype(np.float32)
_RW = np.exp(-((LMAX - np.arange(1, LMAX + 1)) ** 2) / (2.0 * LMAX ** 2)).astype(np.float32)[:, None]


def _sph_sort_kernel(px_ref, py_ref, pz_ref, sxsp_ref, sysp_ref, ftsp_ref):
    x = px_ref[0]
    y = py_ref[0]
    z = pz_ref[0]
    r = jnp.sqrt(x * x + y * y + z * z)
    rho = jnp.sqrt(y * y + z * z)

    def acos(v):
        return jnp.arctan2(jnp.sqrt((1.0 - v) * (1.0 + v)), v)

    theta = acos(jnp.clip(x / r, -1.0, 1.0))
    a = acos(jnp.clip(y / rho, -1.0, 1.0))
    phi = jnp.where(z < 0.0, 2.0 * _PI - a, a) - _PI

    thc = theta.reshape(N, 1)
    thr = theta.reshape(1, N)
    ic = lax.broadcasted_iota(jnp.int32, (N, 1), 0)
    ir = lax.broadcasted_iota(jnp.int32, (1, N), 1)
    less = (thr < thc) | ((thr == thc) & (ir < ic))
    rank = jnp.sum(less.astype(jnp.float32), axis=1, keepdims=True)

    kc = lax.broadcasted_iota(jnp.int32, (N, 1), 0)
    pt = (rank.astype(jnp.int32).reshape(1, N) == kc).astype(jnp.float32)
    vals = jnp.concatenate(
        [theta.reshape(N, 1), phi.reshape(N, 1), r.reshape(N, 1)], axis=1)
    svals = lax.dot(pt, vals, precision=lax.Precision.HIGHEST,
                    preferred_element_type=jnp.float32)

    sxsp_ref[...] = jnp.broadcast_to(svals[:, 0:1], (N, 16)).reshape(1, N, 16)
    sysp_ref[...] = jnp.broadcast_to(svals[:, 1:2], (N, 16)).reshape(1, N, 16)
    ftsp_ref[...] = jnp.broadcast_to(svals[:, 2:3], (N, 16)).reshape(1, N, 16)


_SC_MESH = plsc.VectorSubcoreMesh(core_axis_name="c", subcore_axis_name="s")


@functools.partial(
    pl.kernel,
    mesh=_SC_MESH,
    out_type=[jax.ShapeDtypeStruct((TOTQ,), jnp.float32) for _ in range(6)],
    scratch_types=[
        pltpu.VMEM((N * 16,), jnp.float32),
        pltpu.VMEM((N * 16,), jnp.float32),
        pltpu.VMEM((N * 16,), jnp.float32),
        pltpu.VMEM((QW,), jnp.float32),
        pltpu.VMEM((QW,), jnp.float32),
        pltpu.VMEM((QW,), jnp.float32),
        pltpu.VMEM((QW,), jnp.float32),
        pltpu.VMEM((QW,), jnp.float32),
        pltpu.VMEM((QW,), jnp.float32),
    ],
)
def _sc_knn(sxsp_hbm, sysp_hbm, ftsp_hbm,
            o_f1, o_f2, o_f3, o_d1, o_d2, o_d3,
            sxspv, syspv, ftspv,
            bf1, bf2, bf3, bd1, bd2, bd3):
    b = lax.axis_index("c")
    s = lax.axis_index("s")

    pltpu.sync_copy(sxsp_hbm.at[b], sxspv)
    pltpu.sync_copy(sysp_hbm.at[b], syspv)
    pltpu.sync_copy(ftsp_hbm.at[b], ftspv)

    lanes = lax.iota(jnp.int32, 16)
    scale = jnp.float32(_PI / NLAT)
    inf = jnp.full((16,), jnp.inf, jnp.float32)
    zero = jnp.zeros((16,), jnp.float32)

    def qstep(i, carry_):
        j = i >> 4
        c16 = i & 15
        row = s + 16 * j
        tqs = row.astype(jnp.float32) * scale
        tq = jnp.broadcast_to(tqs, (16,))
        col = c16 * 16 + lanes
        pq = (col.astype(jnp.float32) - np.float32(NLAT)) * scale

        def bstep(k, lh):
            lo, hi = lh
            mid = (lo + hi) >> 1
            c = sxspv[pl.ds(mid * 256, 16)][0] <= tqs
            return (jnp.where(c, mid, lo), jnp.where(c, hi, mid))

        c0, _ = lax.fori_loop(0, 6, bstep, (jnp.int32(0), jnp.int32(NCHUNK)))

        def scan_chunk(c, ks):
            m1, m2, m3, f1, f2, f3 = ks
            for jj in range(16):
                o = c * 256 + jj * 16
                rx = sxspv[pl.ds(o, 16)]
                ry = syspv[pl.ds(o, 16)]
                rf = ftspv[pl.ds(o, 16)]
                dx = tq - rx
                dy = pq - ry
                d2 = dx * dx + dy * dy
                c1 = d2 < m1
                c2 = d2 < m2
                c3 = d2 < m3
                m3 = jnp.where(c2, m2, jnp.where(c3, d2, m3))
                f3 = jnp.where(c2, f2, jnp.where(c3, rf, f3))
                m2 = jnp.where(c1, m1, jnp.where(c2, d2, m2))
                f2 = jnp.where(c1, f1, jnp.where(c2, rf, f2))
                m1 = jnp.where(c1, d2, m1)
                f1 = jnp.where(c1, rf, f1)
            return m1, m2, m3, f1, f2, f3

        ca = jnp.maximum(c0 - 1, 0)
        cb = jnp.minimum(c0 + 2, jnp.int32(NCHUNK))

        @pl.loop(ca, cb, init_carry=(inf, inf, inf, zero, zero, zero))
        def seed_ks(c, ks):
            return scan_chunk(c, ks)

        m3v = seed_ks[2]
        maxm3 = m3v[0]
        for t in range(1, 16):
            maxm3 = jnp.maximum(maxm3, m3v[t])

        def lstep(k, lh):
            lo, hi = lh
            mid = (lo + hi + 1) >> 1
            th = sxspv[pl.ds(jnp.maximum(mid * 16 - 1, 0) * 16, 16)][0]
            g = tqs - th
            p = (mid == 0) | (maxm3 <= g * g)
            return (jnp.where(p, mid, lo), jnp.where(p, hi, mid - 1))

        wl, _ = lax.fori_loop(0, 6, lstep, (jnp.int32(0), ca))

        def rstep(k, lh):
            lo, hi = lh
            mid = (lo + hi) >> 1
            th = sxspv[pl.ds(jnp.minimum(mid * 16, N - 1) * 16, 16)][0]
            g = th - tqs
            p = (mid == NCHUNK) | (maxm3 <= g * g)
            return (jnp.where(p, lo, mid + 1), jnp.where(p, mid, hi))

        _, wr = lax.fori_loop(0, 6, rstep, (cb, jnp.int32(NCHUNK)))

        @pl.loop(wl, ca, init_carry=seed_ks)
        def left_ks(c, ks):
            return scan_chunk(c, ks)

        @pl.loop(cb, wr, init_carry=left_ks)
        def right_ks(c, ks):
            return scan_chunk(c, ks)

        m1, m2, m3, f1, f2, f3 = right_ks

        off = pl.ds(i * 16, 16)
        bd1[off] = m1
        bd2[off] = m2
        bd3[off] = m3
        bf1[off] = f1
        bf2[off] = f2
        bf3[off] = f3
        return carry_

    lax.fori_loop(0, QW // 16, qstep, 0)

    dst = pl.ds((b * 16 + s) * QW, QW)
    pltpu.sync_copy(bf1, o_f1.at[dst])
    pltpu.sync_copy(bf2, o_f2.at[dst])
    pltpu.sync_copy(bf3, o_f3.at[dst])
    pltpu.sync_copy(bd1, o_d1.at[dst])
    pltpu.sync_copy(bd2, o_d2.at[dst])
    pltpu.sync_copy(bd3, o_d3.at[dst])


def _sht_kernel(f1_ref, f2_ref, f3_ref, d1_ref, d2_ref, d3_ref,
                t_ref, cos_ref, wt_ref, rw_ref, out_ref):
    w1 = jnp.sqrt(jnp.maximum(d1_ref[...], 1e-12))
    w2 = jnp.sqrt(jnp.maximum(d2_ref[...], 1e-12))
    w3 = jnp.sqrt(jnp.maximum(d3_ref[...], 1e-12))
    x = (f1_ref[...] * w1 + f2_ref[...] * w2 + f3_ref[...] * w3) / (w1 + w2 + w3)
    xr = lax.dot(x, cos_ref[...], precision=lax.Precision.HIGHEST,
                 preferred_element_type=jnp.float32)
    xr = xr * np.float32(2.0 * _PI / NLON)
    wt = wt_ref[...]
    rw = rw_ref[...]
    loss = jnp.float32(0.0)
    for b in range(B):
        xb = xr[b * NLAT:(b + 1) * NLAT]
        cb = jnp.sum(wt * xb[:, None, :], axis=0)
        resid = cb - t_ref[b]
        loss = loss + jnp.sum(resid * resid * rw)
    out_ref[...] = (loss / B).reshape(1, 1)


def kernel(pred, target_coeffs):
    px = pred[:, :, 0].reshape(B, 1, N)
    py = pred[:, :, 1].reshape(B, 1, N)
    pz = pred[:, :, 2].reshape(B, 1, N)

    sxsp, sysp, ftsp = pl.pallas_call(
        _sph_sort_kernel,
        grid=(B,),
        in_specs=[pl.BlockSpec((1, 1, N), lambda b: (b, 0, 0))] * 3,
        out_specs=[pl.BlockSpec((1, N, 16), lambda b: (b, 0, 0))] * 3,
        out_shape=[jax.ShapeDtypeStruct((B, N, 16), jnp.float32)] * 3,
    )(px, py, pz)

    f1, f2, f3, d1, d2, d3 = _sc_knn(
        sxsp.reshape(B, N * 16), sysp.reshape(B, N * 16), ftsp.reshape(B, N * 16))

    shp = (B * NLAT, NLON)
    loss = pl.pallas_call(
        _sht_kernel,
        out_shape=jax.ShapeDtypeStruct((1, 1), jnp.float32),
    )(f1.reshape(shp), f2.reshape(shp), f3.reshape(shp),
      d1.reshape(shp), d2.reshape(shp), d3.reshape(shp),
      target_coeffs, jnp.asarray(_COS), jnp.asarray(_WTP), jnp.asarray(_RW))
    return loss[0, 0]


# speedup vs baseline: 2.6330x; 1.9998x over previous
import functools
import math

import jax
import jax.numpy as jnp
import numpy as np
from jax import lax
from jax.experimental import pallas as pl
from jax.experimental.pallas import tpu as pltpu
from jax.experimental.pallas import tpu_sc as plsc

NLAT = 128
NLON = 256
LMAX = 50
MMAX = 50
N = 1024
B = 2
G = NLAT * NLON
TOTQ = B * G
QW = 2048
NCHUNK = N // 16

_PI = math.pi


def _cc_quad_weights(n):
    tj = np.pi * np.arange(n) / (n - 1)
    x = np.cos(tj)
    Nn = n - 1
    w = np.zeros(n)
    for j in range(n):
        tmp = 0.0
        for k in range(1, Nn // 2 + 1):
            bk = 1.0 if 2 * k == Nn else 2.0
            tmp += bk / (4.0 * k * k - 1.0) * np.cos(2.0 * k * tj[j])
        wj = 1.0 - tmp
        wj = wj / Nn if (j == 0 or j == Nn) else 2.0 * wj / Nn
        w[j] = wj
    return x, w


def _legpoly(mmax, lmax, x):
    nlat = x.shape[0]
    pct = np.zeros((mmax, lmax, nlat))
    sint = np.sqrt(np.clip(1.0 - x * x, 0.0, None))
    pmm = np.full(nlat, math.sqrt(1.0 / (4.0 * math.pi)))
    for m in range(mmax):
        if m > 0:
            pmm = -math.sqrt((2.0 * m + 1.0) / (2.0 * m)) * sint * pmm
        if m < lmax:
            pct[m, m] = pmm
        if m + 1 < lmax:
            pct[m, m + 1] = math.sqrt(2.0 * m + 3.0) * x * pmm
        for l in range(m + 2, lmax):
            a = math.sqrt((4.0 * l * l - 1.0) / (l * l - m * m))
            b = math.sqrt((((l - 1.0) ** 2) - m * m) / (4.0 * (l - 1.0) ** 2 - 1.0))
            pct[m, l] = a * (x * pct[m, l - 1] - b * pct[m, l - 2])
    return pct


_COST, _WQ = _cc_quad_weights(NLAT)
_SHT_W = (_legpoly(MMAX, LMAX, _COST) * _WQ[None, None, :]).astype(np.float32)
_WT = np.ascontiguousarray(np.transpose(_SHT_W, (2, 1, 0)))
_ROWPERM = np.array([(q // 8) + 16 * (q % 8) for q in range(NLAT)])
_WTP = np.ascontiguousarray(_WT[_ROWPERM])
_j = np.arange(NLON)[:, None].astype(np.float64)
_m = np.arange(MMAX)[None, :].astype(np.float64)
_COS = np.cos(2.0 * np.pi * _j * _m / NLON).astype(np.float32)
_RW = np.exp(-((LMAX - np.arange(1, LMAX + 1)) ** 2) / (2.0 * LMAX ** 2)).astype(np.float32)[:, None]


def _sph_sort_kernel(px_ref, py_ref, pz_ref, sxsp_ref, sysp_ref, ftsp_ref):
    x = px_ref[0]
    y = py_ref[0]
    z = pz_ref[0]
    r = jnp.sqrt(x * x + y * y + z * z)
    rho = jnp.sqrt(y * y + z * z)

    def acos(v):
        return jnp.arctan2(jnp.sqrt((1.0 - v) * (1.0 + v)), v)

    theta = acos(jnp.clip(x / r, -1.0, 1.0))
    a = acos(jnp.clip(y / rho, -1.0, 1.0))
    phi = jnp.where(z < 0.0, 2.0 * _PI - a, a) - _PI

    def outer(row_a, row_b):
        return lax.dot_general(
            row_a, row_b, (((0,), (0,)), ((), ())),
            precision=lax.Precision.HIGHEST,
            preferred_element_type=jnp.float32)

    ones_row = jnp.ones((1, N), jnp.float32)
    thc = outer(theta, ones_row)
    ic = lax.broadcasted_iota(jnp.int32, (N, N), 0)
    ir = lax.broadcasted_iota(jnp.int32, (N, N), 1)
    m = (thc < theta) | ((thc == theta) & (ic < ir))
    rank_row = jnp.sum(m.astype(jnp.float32), axis=0, keepdims=True)

    rankc = outer(rank_row, ones_row).astype(jnp.int32)
    pt2 = (rankc == ir).astype(jnp.float32)
    vals_rows = jnp.concatenate([theta, phi, r], axis=0)
    svals = lax.dot_general(
        vals_rows, pt2, (((1,), (0,)), ((), ())),
        precision=lax.Precision.HIGHEST,
        preferred_element_type=jnp.float32)

    ones16 = jnp.ones((1, 16), jnp.float32)
    sxsp_ref[...] = outer(svals[0:1], ones16).reshape(1, N, 16)
    sysp_ref[...] = outer(svals[1:2], ones16).reshape(1, N, 16)
    ftsp_ref[...] = outer(svals[2:3], ones16).reshape(1, N, 16)


_SC_MESH = plsc.VectorSubcoreMesh(core_axis_name="c", subcore_axis_name="s")


@functools.partial(
    pl.kernel,
    mesh=_SC_MESH,
    out_type=[jax.ShapeDtypeStruct((TOTQ,), jnp.float32) for _ in range(6)],
    scratch_types=[
        pltpu.VMEM((N * 16,), jnp.float32),
        pltpu.VMEM((N * 16,), jnp.float32),
        pltpu.VMEM((N * 16,), jnp.float32),
        pltpu.VMEM((QW,), jnp.float32),
        pltpu.VMEM((QW,), jnp.float32),
        pltpu.VMEM((QW,), jnp.float32),
        pltpu.VMEM((QW,), jnp.float32),
        pltpu.VMEM((QW,), jnp.float32),
        pltpu.VMEM((QW,), jnp.float32),
    ],
)
def _sc_knn(sxsp_hbm, sysp_hbm, ftsp_hbm,
            o_f1, o_f2, o_f3, o_d1, o_d2, o_d3,
            sxspv, syspv, ftspv,
            bf1, bf2, bf3, bd1, bd2, bd3):
    b = lax.axis_index("c")
    s = lax.axis_index("s")

    pltpu.sync_copy(sxsp_hbm.at[b], sxspv)
    pltpu.sync_copy(sysp_hbm.at[b], syspv)
    pltpu.sync_copy(ftsp_hbm.at[b], ftspv)

    lanes = lax.iota(jnp.int32, 16)
    scale = jnp.float32(_PI / NLAT)
    inf = jnp.full((16,), jnp.inf, jnp.float32)
    zero = jnp.zeros((16,), jnp.float32)

    def qstep(i, carry_):
        j = i >> 4
        c16 = i & 15
        row = s + 16 * j
        tqs = row.astype(jnp.float32) * scale
        tq = jnp.broadcast_to(tqs, (16,))
        col = c16 * 16 + lanes
        pq = (col.astype(jnp.float32) - np.float32(NLAT)) * scale

        def bstep(k, lh):
            lo, hi = lh
            mid = (lo + hi) >> 1
            c = sxspv[pl.ds(mid * 256, 16)][0] <= tqs
            return (jnp.where(c, mid, lo), jnp.where(c, hi, mid))

        c0, _ = lax.fori_loop(0, 6, bstep, (jnp.int32(0), jnp.int32(NCHUNK)))

        def scan_chunk(c, ks):
            m1, m2, m3, f1, f2, f3 = ks
            for jj in range(16):
                o = c * 256 + jj * 16
                rx = sxspv[pl.ds(o, 16)]
                ry = syspv[pl.ds(o, 16)]
                rf = ftspv[pl.ds(o, 16)]
                dx = tq - rx
                dy = pq - ry
                d2 = dx * dx + dy * dy
                c1 = d2 < m1
                c2 = d2 < m2
                c3 = d2 < m3
                m3 = jnp.where(c2, m2, jnp.where(c3, d2, m3))
                f3 = jnp.where(c2, f2, jnp.where(c3, rf, f3))
                m2 = jnp.where(c1, m1, jnp.where(c2, d2, m2))
                f2 = jnp.where(c1, f1, jnp.where(c2, rf, f2))
                m1 = jnp.where(c1, d2, m1)
                f1 = jnp.where(c1, rf, f1)
            return m1, m2, m3, f1, f2, f3

        ca = jnp.maximum(c0 - 1, 0)
        cb = jnp.minimum(c0 + 2, jnp.int32(NCHUNK))

        @pl.loop(ca, cb, init_carry=(inf, inf, inf, zero, zero, zero))
        def seed_ks(c, ks):
            return scan_chunk(c, ks)

        m3v = seed_ks[2]
        maxm3 = m3v[0]
        for t in range(1, 16):
            maxm3 = jnp.maximum(maxm3, m3v[t])

        def lstep(k, lh):
            lo, hi = lh
            mid = (lo + hi + 1) >> 1
            th = sxspv[pl.ds(jnp.maximum(mid * 16 - 1, 0) * 16, 16)][0]
            g = tqs - th
            p = (mid == 0) | (maxm3 <= g * g)
            return (jnp.where(p, mid, lo), jnp.where(p, hi, mid - 1))

        wl, _ = lax.fori_loop(0, 6, lstep, (jnp.int32(0), ca))

        def rstep(k, lh):
            lo, hi = lh
            mid = (lo + hi) >> 1
            th = sxspv[pl.ds(jnp.minimum(mid * 16, N - 1) * 16, 16)][0]
            g = th - tqs
            p = (mid == NCHUNK) | (maxm3 <= g * g)
            return (jnp.where(p, lo, mid + 1), jnp.where(p, mid, hi))

        _, wr = lax.fori_loop(0, 6, rstep, (cb, jnp.int32(NCHUNK)))

        @pl.loop(wl, ca, init_carry=seed_ks)
        def left_ks(c, ks):
            return scan_chunk(c, ks)

        @pl.loop(cb, wr, init_carry=left_ks)
        def right_ks(c, ks):
            return scan_chunk(c, ks)

        m1, m2, m3, f1, f2, f3 = right_ks

        off = pl.ds(i * 16, 16)
        bd1[off] = m1
        bd2[off] = m2
        bd3[off] = m3
        bf1[off] = f1
        bf2[off] = f2
        bf3[off] = f3
        return carry_

    lax.fori_loop(0, QW // 16, qstep, 0)

    dst = pl.ds((b * 16 + s) * QW, QW)
    pltpu.sync_copy(bf1, o_f1.at[dst])
    pltpu.sync_copy(bf2, o_f2.at[dst])
    pltpu.sync_copy(bf3, o_f3.at[dst])
    pltpu.sync_copy(bd1, o_d1.at[dst])
    pltpu.sync_copy(bd2, o_d2.at[dst])
    pltpu.sync_copy(bd3, o_d3.at[dst])


def _sht_kernel(f1_ref, f2_ref, f3_ref, d1_ref, d2_ref, d3_ref,
                t_ref, cos_ref, wt_ref, rw_ref, out_ref):
    w1 = jnp.sqrt(jnp.maximum(d1_ref[...], 1e-12))
    w2 = jnp.sqrt(jnp.maximum(d2_ref[...], 1e-12))
    w3 = jnp.sqrt(jnp.maximum(d3_ref[...], 1e-12))
    x = (f1_ref[...] * w1 + f2_ref[...] * w2 + f3_ref[...] * w3) / (w1 + w2 + w3)
    xr = lax.dot(x, cos_ref[...], precision=lax.Precision.HIGHEST,
                 preferred_element_type=jnp.float32)
    xr = xr * np.float32(2.0 * _PI / NLON)
    wt = wt_ref[...]
    rw = rw_ref[...]
    loss = jnp.float32(0.0)
    for b in range(B):
        xb = xr[b * NLAT:(b + 1) * NLAT]
        cb = jnp.sum(wt * xb[:, None, :], axis=0)
        resid = cb - t_ref[b]
        loss = loss + jnp.sum(resid * resid * rw)
    out_ref[...] = (loss / B).reshape(1, 1)


def kernel(pred, target_coeffs):
    px = pred[:, :, 0].reshape(B, 1, N)
    py = pred[:, :, 1].reshape(B, 1, N)
    pz = pred[:, :, 2].reshape(B, 1, N)

    sxsp, sysp, ftsp = pl.pallas_call(
        _sph_sort_kernel,
        grid=(B,),
        in_specs=[pl.BlockSpec((1, 1, N), lambda b: (b, 0, 0))] * 3,
        out_specs=[pl.BlockSpec((1, N, 16), lambda b: (b, 0, 0))] * 3,
        out_shape=[jax.ShapeDtypeStruct((B, N, 16), jnp.float32)] * 3,
    )(px, py, pz)

    f1, f2, f3, d1, d2, d3 = _sc_knn(
        sxsp.reshape(B, N * 16), sysp.reshape(B, N * 16), ftsp.reshape(B, N * 16))

    shp = (B * NLAT, NLON)
    loss = pl.pallas_call(
        _sht_kernel,
        out_shape=jax.ShapeDtypeStruct((1, 1), jnp.float32),
    )(f1.reshape(shp), f2.reshape(shp), f3.reshape(shp),
      d1.reshape(shp), d2.reshape(shp), d3.reshape(shp),
      target_coeffs, jnp.asarray(_COS), jnp.asarray(_WTP), jnp.asarray(_RW))
    return loss[0, 0]


# speedup vs baseline: 3.2022x; 1.2162x over previous
import functools
import math

import jax
import jax.numpy as jnp
import numpy as np
from jax import lax
from jax.experimental import pallas as pl
from jax.experimental.pallas import tpu as pltpu
from jax.experimental.pallas import tpu_sc as plsc

NLAT = 128
NLON = 256
LMAX = 50
MMAX = 50
N = 1024
B = 2
G = NLAT * NLON
TOTQ = B * G
QW = 2048
NCHUNK = N // 16

_PI = math.pi


def _cc_quad_weights(n):
    tj = np.pi * np.arange(n) / (n - 1)
    x = np.cos(tj)
    Nn = n - 1
    w = np.zeros(n)
    for j in range(n):
        tmp = 0.0
        for k in range(1, Nn // 2 + 1):
            bk = 1.0 if 2 * k == Nn else 2.0
            tmp += bk / (4.0 * k * k - 1.0) * np.cos(2.0 * k * tj[j])
        wj = 1.0 - tmp
        wj = wj / Nn if (j == 0 or j == Nn) else 2.0 * wj / Nn
        w[j] = wj
    return x, w


def _legpoly(mmax, lmax, x):
    nlat = x.shape[0]
    pct = np.zeros((mmax, lmax, nlat))
    sint = np.sqrt(np.clip(1.0 - x * x, 0.0, None))
    pmm = np.full(nlat, math.sqrt(1.0 / (4.0 * math.pi)))
    for m in range(mmax):
        if m > 0:
            pmm = -math.sqrt((2.0 * m + 1.0) / (2.0 * m)) * sint * pmm
        if m < lmax:
            pct[m, m] = pmm
        if m + 1 < lmax:
            pct[m, m + 1] = math.sqrt(2.0 * m + 3.0) * x * pmm
        for l in range(m + 2, lmax):
            a = math.sqrt((4.0 * l * l - 1.0) / (l * l - m * m))
            b = math.sqrt((((l - 1.0) ** 2) - m * m) / (4.0 * (l - 1.0) ** 2 - 1.0))
            pct[m, l] = a * (x * pct[m, l - 1] - b * pct[m, l - 2])
    return pct


_COST, _WQ = _cc_quad_weights(NLAT)
_SHT_W = (_legpoly(MMAX, LMAX, _COST) * _WQ[None, None, :]).astype(np.float32)
_WT = np.ascontiguousarray(np.transpose(_SHT_W, (2, 1, 0)))
_ROWPERM = np.array([(q // 8) + 16 * (q % 8) for q in range(NLAT)])
_WTP = np.ascontiguousarray(_WT[_ROWPERM])
_j = np.arange(NLON)[:, None].astype(np.float64)
_m = np.arange(MMAX)[None, :].astype(np.float64)
_COS = np.cos(2.0 * np.pi * _j * _m / NLON).astype(np.float32)
_RW = np.exp(-((LMAX - np.arange(1, LMAX + 1)) ** 2) / (2.0 * LMAX ** 2)).astype(np.float32)[:, None]


def _sph_sort_kernel(px_ref, py_ref, pz_ref, sxsp_ref, sysp_ref, ftsp_ref):
    x = px_ref[0]
    y = py_ref[0]
    z = pz_ref[0]
    r = jnp.sqrt(x * x + y * y + z * z)
    rho = jnp.sqrt(y * y + z * z)

    def acos(v):
        return jnp.arctan2(jnp.sqrt((1.0 - v) * (1.0 + v)), v)

    theta = acos(jnp.clip(x / r, -1.0, 1.0))
    a = acos(jnp.clip(y / rho, -1.0, 1.0))
    phi = jnp.where(z < 0.0, 2.0 * _PI - a, a) - _PI

    def outer(row_a, row_b):
        return lax.dot_general(
            row_a, row_b, (((0,), (0,)), ((), ())),
            precision=lax.Precision.HIGHEST,
            preferred_element_type=jnp.float32)

    ones_row = jnp.ones((1, N), jnp.float32)
    thc = outer(theta, ones_row)
    ic = lax.broadcasted_iota(jnp.int32, (N, N), 0)
    ir = lax.broadcasted_iota(jnp.int32, (N, N), 1)
    m = (thc < theta) | ((thc == theta) & (ic < ir))
    rank_row = jnp.sum(m.astype(jnp.float32), axis=0, keepdims=True)

    rankc = outer(rank_row, ones_row).astype(jnp.int32)
    pt2 = (rankc == ir).astype(jnp.float32)
    vals_rows = jnp.concatenate([theta, phi, r], axis=0)
    svals = lax.dot_general(
        vals_rows, pt2, (((1,), (0,)), ((), ())),
        precision=lax.Precision.HIGHEST,
        preferred_element_type=jnp.float32)

    ones16 = jnp.ones((1, 16), jnp.float32)
    sxsp_ref[...] = outer(svals[0:1], ones16).reshape(1, N, 16)
    sysp_ref[...] = outer(svals[1:2], ones16).reshape(1, N, 16)
    ftsp_ref[...] = outer(svals[2:3], ones16).reshape(1, N, 16)


_SC_MESH = plsc.VectorSubcoreMesh(core_axis_name="c", subcore_axis_name="s")


@functools.partial(
    pl.kernel,
    mesh=_SC_MESH,
    out_type=[jax.ShapeDtypeStruct((TOTQ,), jnp.float32) for _ in range(6)],
    scratch_types=[
        pltpu.VMEM((N * 16,), jnp.float32),
        pltpu.VMEM((N * 16,), jnp.float32),
        pltpu.VMEM((N * 16,), jnp.float32),
        pltpu.VMEM((QW,), jnp.float32),
        pltpu.VMEM((QW,), jnp.float32),
        pltpu.VMEM((QW,), jnp.float32),
        pltpu.VMEM((QW,), jnp.float32),
        pltpu.VMEM((QW,), jnp.float32),
        pltpu.VMEM((QW,), jnp.float32),
    ],
)
def _sc_knn(sxsp_hbm, sysp_hbm, ftsp_hbm,
            o_f1, o_f2, o_f3, o_d1, o_d2, o_d3,
            sxspv, syspv, ftspv,
            bf1, bf2, bf3, bd1, bd2, bd3):
    b = lax.axis_index("c")
    s = lax.axis_index("s")

    pltpu.sync_copy(sxsp_hbm.at[b], sxspv)
    pltpu.sync_copy(sysp_hbm.at[b], syspv)
    pltpu.sync_copy(ftsp_hbm.at[b], ftspv)

    lanes = lax.iota(jnp.int32, 16)
    scale = jnp.float32(_PI / NLAT)
    inf = jnp.full((16,), jnp.inf, jnp.float32)
    zero = jnp.zeros((16,), jnp.float32)

    def qstep(i, carry_):
        j = i >> 4
        c16 = i & 15
        row = s + 16 * j
        tqs = row.astype(jnp.float32) * scale
        tq = jnp.broadcast_to(tqs, (16,))
        col = c16 * 16 + lanes
        pq = (col.astype(jnp.float32) - np.float32(NLAT)) * scale

        def bstep(k, lh):
            lo, hi = lh
            mid = (lo + hi) >> 1
            c = sxspv[pl.ds(mid * 256, 16)][0] <= tqs
            return (jnp.where(c, mid, lo), jnp.where(c, hi, mid))

        c0, _ = lax.fori_loop(0, 6, bstep, (jnp.int32(0), jnp.int32(NCHUNK)))

        def scan_chunk(c, ks):
            m1, m2, m3, f1, f2, f3 = ks
            for jj in range(16):
                o = c * 256 + jj * 16
                rx = sxspv[pl.ds(o, 16)]
                ry = syspv[pl.ds(o, 16)]
                rf = ftspv[pl.ds(o, 16)]
                dx = tq - rx
                dy = pq - ry
                d2 = dx * dx + dy * dy
                c1 = d2 < m1
                c2 = d2 < m2
                c3 = d2 < m3
                m3 = jnp.where(c2, m2, jnp.where(c3, d2, m3))
                f3 = jnp.where(c2, f2, jnp.where(c3, rf, f3))
                m2 = jnp.where(c1, m1, jnp.where(c2, d2, m2))
                f2 = jnp.where(c1, f1, jnp.where(c2, rf, f2))
                m1 = jnp.where(c1, d2, m1)
                f1 = jnp.where(c1, rf, f1)
            return m1, m2, m3, f1, f2, f3

        ca = jnp.maximum(c0 - 2, 0)
        cb = jnp.minimum(c0 + 3, jnp.int32(NCHUNK))

        @pl.loop(ca, cb, init_carry=(inf, inf, inf, zero, zero, zero))
        def seed_ks(c, ks):
            return scan_chunk(c, ks)

        m3v = seed_ks[2]
        maxm3 = m3v[0]
        for t in range(1, 16):
            maxm3 = jnp.maximum(maxm3, m3v[t])

        def lstep(k, lh):
            lo, hi = lh
            mid = (lo + hi + 1) >> 1
            th = sxspv[pl.ds(jnp.maximum(mid * 16 - 1, 0) * 16, 16)][0]
            g = tqs - th
            p = (mid == 0) | (maxm3 <= g * g)
            return (jnp.where(p, mid, lo), jnp.where(p, hi, mid - 1))

        wl, _ = lax.fori_loop(0, 6, lstep, (jnp.int32(0), ca))

        def rstep(k, lh):
            lo, hi = lh
            mid = (lo + hi) >> 1
            th = sxspv[pl.ds(jnp.minimum(mid * 16, N - 1) * 16, 16)][0]
            g = th - tqs
            p = (mid == NCHUNK) | (maxm3 <= g * g)
            return (jnp.where(p, lo, mid + 1), jnp.where(p, mid, hi))

        _, wr = lax.fori_loop(0, 6, rstep, (cb, jnp.int32(NCHUNK)))

        @pl.loop(wl, ca, init_carry=seed_ks)
        def left_ks(c, ks):
            return scan_chunk(c, ks)

        @pl.loop(cb, wr, init_carry=left_ks)
        def right_ks(c, ks):
            return scan_chunk(c, ks)

        m1, m2, m3, f1, f2, f3 = right_ks

        off = pl.ds(i * 16, 16)
        bd1[off] = m1
        bd2[off] = m2
        bd3[off] = m3
        bf1[off] = f1
        bf2[off] = f2
        bf3[off] = f3
        return carry_

    lax.fori_loop(0, QW // 16, qstep, 0)

    dst = pl.ds((b * 16 + s) * QW, QW)
    pltpu.sync_copy(bf1, o_f1.at[dst])
    pltpu.sync_copy(bf2, o_f2.at[dst])
    pltpu.sync_copy(bf3, o_f3.at[dst])
    pltpu.sync_copy(bd1, o_d1.at[dst])
    pltpu.sync_copy(bd2, o_d2.at[dst])
    pltpu.sync_copy(bd3, o_d3.at[dst])


def _sht_kernel(f1_ref, f2_ref, f3_ref, d1_ref, d2_ref, d3_ref,
                t_ref, cos_ref, wt_ref, rw_ref, out_ref):
    w1 = jnp.sqrt(jnp.maximum(d1_ref[...], 1e-12))
    w2 = jnp.sqrt(jnp.maximum(d2_ref[...], 1e-12))
    w3 = jnp.sqrt(jnp.maximum(d3_ref[...], 1e-12))
    x = (f1_ref[...] * w1 + f2_ref[...] * w2 + f3_ref[...] * w3) / (w1 + w2 + w3)
    xr = lax.dot(x, cos_ref[...], precision=lax.Precision.HIGHEST,
                 preferred_element_type=jnp.float32)
    xr = xr * np.float32(2.0 * _PI / NLON)
    wt = wt_ref[...]
    rw = rw_ref[...]
    loss = jnp.float32(0.0)
    for b in range(B):
        xb = xr[b * NLAT:(b + 1) * NLAT]
        cb = jnp.sum(wt * xb[:, None, :], axis=0)
        resid = cb - t_ref[b]
        loss = loss + jnp.sum(resid * resid * rw)
    out_ref[...] = (loss / B).reshape(1, 1)


def kernel(pred, target_coeffs):
    px = pred[:, :, 0].reshape(B, 1, N)
    py = pred[:, :, 1].reshape(B, 1, N)
    pz = pred[:, :, 2].reshape(B, 1, N)

    sxsp, sysp, ftsp = pl.pallas_call(
        _sph_sort_kernel,
        grid=(B,),
        in_specs=[pl.BlockSpec((1, 1, N), lambda b: (b, 0, 0))] * 3,
        out_specs=[pl.BlockSpec((1, N, 16), lambda b: (b, 0, 0))] * 3,
        out_shape=[jax.ShapeDtypeStruct((B, N, 16), jnp.float32)] * 3,
    )(px, py, pz)

    f1, f2, f3, d1, d2, d3 = _sc_knn(
        sxsp.reshape(B, N * 16), sysp.reshape(B, N * 16), ftsp.reshape(B, N * 16))

    shp = (B * NLAT, NLON)
    loss = pl.pallas_call(
        _sht_kernel,
        out_shape=jax.ShapeDtypeStruct((1, 1), jnp.float32),
    )(f1.reshape(shp), f2.reshape(shp), f3.reshape(shp),
      d1.reshape(shp), d2.reshape(shp), d3.reshape(shp),
      target_coeffs, jnp.asarray(_COS), jnp.asarray(_WTP), jnp.asarray(_RW))
    return loss[0, 0]


# speedup vs baseline: 3.3695x; 1.0522x over previous
import functools
import math

import jax
import jax.numpy as jnp
import numpy as np
from jax import lax
from jax.experimental import pallas as pl
from jax.experimental.pallas import tpu as pltpu
from jax.experimental.pallas import tpu_sc as plsc

NLAT = 128
NLON = 256
LMAX = 50
MMAX = 50
N = 1024
B = 2
G = NLAT * NLON
TOTQ = B * G
QW = 2048
NCHUNK = N // 16

_PI = math.pi


def _cc_quad_weights(n):
    tj = np.pi * np.arange(n) / (n - 1)
    x = np.cos(tj)
    Nn = n - 1
    w = np.zeros(n)
    for j in range(n):
        tmp = 0.0
        for k in range(1, Nn // 2 + 1):
            bk = 1.0 if 2 * k == Nn else 2.0
            tmp += bk / (4.0 * k * k - 1.0) * np.cos(2.0 * k * tj[j])
        wj = 1.0 - tmp
        wj = wj / Nn if (j == 0 or j == Nn) else 2.0 * wj / Nn
        w[j] = wj
    return x, w


def _legpoly(mmax, lmax, x):
    nlat = x.shape[0]
    pct = np.zeros((mmax, lmax, nlat))
    sint = np.sqrt(np.clip(1.0 - x * x, 0.0, None))
    pmm = np.full(nlat, math.sqrt(1.0 / (4.0 * math.pi)))
    for m in range(mmax):
        if m > 0:
            pmm = -math.sqrt((2.0 * m + 1.0) / (2.0 * m)) * sint * pmm
        if m < lmax:
            pct[m, m] = pmm
        if m + 1 < lmax:
            pct[m, m + 1] = math.sqrt(2.0 * m + 3.0) * x * pmm
        for l in range(m + 2, lmax):
            a = math.sqrt((4.0 * l * l - 1.0) / (l * l - m * m))
            b = math.sqrt((((l - 1.0) ** 2) - m * m) / (4.0 * (l - 1.0) ** 2 - 1.0))
            pct[m, l] = a * (x * pct[m, l - 1] - b * pct[m, l - 2])
    return pct


_COST, _WQ = _cc_quad_weights(NLAT)
_SHT_W = (_legpoly(MMAX, LMAX, _COST) * _WQ[None, None, :]).astype(np.float32)
_WT = np.ascontiguousarray(np.transpose(_SHT_W, (2, 1, 0)))
_ROWPERM = np.array([(q // 8) + 16 * (q % 8) for q in range(NLAT)])
_WTP = np.ascontiguousarray(_WT[_ROWPERM])
_j = np.arange(NLON)[:, None].astype(np.float64)
_m = np.arange(MMAX)[None, :].astype(np.float64)
_COS = np.cos(2.0 * np.pi * _j * _m / NLON).astype(np.float32)
_RW = np.exp(-((LMAX - np.arange(1, LMAX + 1)) ** 2) / (2.0 * LMAX ** 2)).astype(np.float32)[:, None]


def _sph_sort_kernel(px_ref, py_ref, pz_ref, sxsp_ref, sysp_ref, ftsp_ref):
    x = px_ref[0]
    y = py_ref[0]
    z = pz_ref[0]
    r = jnp.sqrt(x * x + y * y + z * z)
    rho = jnp.sqrt(y * y + z * z)

    def acos(v):
        return jnp.arctan2(jnp.sqrt((1.0 - v) * (1.0 + v)), v)

    theta = acos(jnp.clip(x / r, -1.0, 1.0))
    a = acos(jnp.clip(y / rho, -1.0, 1.0))
    phi = jnp.where(z < 0.0, 2.0 * _PI - a, a) - _PI

    def outer(row_a, row_b):
        return lax.dot_general(
            row_a, row_b, (((0,), (0,)), ((), ())),
            precision=lax.Precision.HIGHEST,
            preferred_element_type=jnp.float32)

    ones_row = jnp.ones((1, N), jnp.float32)
    thc = outer(theta, ones_row)
    ic = lax.broadcasted_iota(jnp.int32, (N, N), 0)
    ir = lax.broadcasted_iota(jnp.int32, (N, N), 1)
    m = (thc < theta) | ((thc == theta) & (ic < ir))
    rank_row = jnp.sum(m.astype(jnp.float32), axis=0, keepdims=True)

    rankc = outer(rank_row, ones_row).astype(jnp.int32)
    pt2 = (rankc == ir).astype(jnp.float32)
    vals_rows = jnp.concatenate([theta, phi, r], axis=0)
    svals = lax.dot_general(
        vals_rows, pt2, (((1,), (0,)), ((), ())),
        precision=lax.Precision.HIGHEST,
        preferred_element_type=jnp.float32)

    ones16 = jnp.ones((1, 16), jnp.float32)
    sxsp_ref[...] = outer(svals[0:1], ones16).reshape(1, N, 16)
    sysp_ref[...] = outer(svals[1:2], ones16).reshape(1, N, 16)
    ftsp_ref[...] = outer(svals[2:3], ones16).reshape(1, N, 16)


_SC_MESH = plsc.VectorSubcoreMesh(core_axis_name="c", subcore_axis_name="s")


@functools.partial(
    pl.kernel,
    mesh=_SC_MESH,
    out_type=[jax.ShapeDtypeStruct((TOTQ,), jnp.float32) for _ in range(6)],
    scratch_types=[
        pltpu.VMEM((N * 16,), jnp.float32),
        pltpu.VMEM((N * 16,), jnp.float32),
        pltpu.VMEM((N * 16,), jnp.float32),
        pltpu.VMEM((QW,), jnp.float32),
        pltpu.VMEM((QW,), jnp.float32),
        pltpu.VMEM((QW,), jnp.float32),
        pltpu.VMEM((QW,), jnp.float32),
        pltpu.VMEM((QW,), jnp.float32),
        pltpu.VMEM((QW,), jnp.float32),
    ],
)
def _sc_knn(sxsp_hbm, sysp_hbm, ftsp_hbm,
            o_f1, o_f2, o_f3, o_d1, o_d2, o_d3,
            sxspv, syspv, ftspv,
            bf1, bf2, bf3, bd1, bd2, bd3):
    b = lax.axis_index("c")
    s = lax.axis_index("s")

    pltpu.sync_copy(sxsp_hbm.at[b], sxspv)
    pltpu.sync_copy(sysp_hbm.at[b], syspv)
    pltpu.sync_copy(ftsp_hbm.at[b], ftspv)

    lanes = lax.iota(jnp.int32, 16)
    scale = jnp.float32(_PI / NLAT)
    inf = jnp.full((16,), jnp.inf, jnp.float32)
    zero = jnp.zeros((16,), jnp.float32)

    def qstep(i, carry_):
        j = i >> 4
        c16 = i & 15
        row = s + 16 * j
        tqs = row.astype(jnp.float32) * scale
        tq = jnp.broadcast_to(tqs, (16,))
        col = c16 * 16 + lanes
        pq = (col.astype(jnp.float32) - np.float32(NLAT)) * scale

        def bstep(k, lh):
            lo, hi = lh
            mid = (lo + hi) >> 1
            c = sxspv[pl.ds(mid * 256, 16)][0] <= tqs
            return (jnp.where(c, mid, lo), jnp.where(c, hi, mid))

        c0, _ = lax.fori_loop(0, 6, bstep, (jnp.int32(0), jnp.int32(NCHUNK)))

        def scan_chunk(c, ks):
            m1, m2, m3, f1, f2, f3 = ks
            for jj in range(16):
                o = c * 256 + jj * 16
                rx = sxspv[pl.ds(o, 16)]
                ry = syspv[pl.ds(o, 16)]
                rf = ftspv[pl.ds(o, 16)]
                dx = tq - rx
                dy = pq - ry
                d2 = dx * dx + dy * dy
                c1 = d2 < m1
                c2 = d2 < m2
                c3 = d2 < m3
                m3 = jnp.where(c2, m2, jnp.where(c3, d2, m3))
                f3 = jnp.where(c2, f2, jnp.where(c3, rf, f3))
                m2 = jnp.where(c1, m1, jnp.where(c2, d2, m2))
                f2 = jnp.where(c1, f1, jnp.where(c2, rf, f2))
                m1 = jnp.where(c1, d2, m1)
                f1 = jnp.where(c1, rf, f1)
            return m1, m2, m3, f1, f2, f3

        ca = jnp.maximum(c0 - 3, 0)
        cb = jnp.minimum(c0 + 4, jnp.int32(NCHUNK))

        @pl.loop(ca, cb, init_carry=(inf, inf, inf, zero, zero, zero))
        def seed_ks(c, ks):
            return scan_chunk(c, ks)

        m3v = seed_ks[2]
        maxm3 = m3v[0]
        for t in range(1, 16):
            maxm3 = jnp.maximum(maxm3, m3v[t])

        def lstep(k, lh):
            lo, hi = lh
            mid = (lo + hi + 1) >> 1
            th = sxspv[pl.ds(jnp.maximum(mid * 16 - 1, 0) * 16, 16)][0]
            g = tqs - th
            p = (mid == 0) | (maxm3 <= g * g)
            return (jnp.where(p, mid, lo), jnp.where(p, hi, mid - 1))

        wl, _ = lax.fori_loop(0, 6, lstep, (jnp.int32(0), ca))

        def rstep(k, lh):
            lo, hi = lh
            mid = (lo + hi) >> 1
            th = sxspv[pl.ds(jnp.minimum(mid * 16, N - 1) * 16, 16)][0]
            g = th - tqs
            p = (mid == NCHUNK) | (maxm3 <= g * g)
            return (jnp.where(p, lo, mid + 1), jnp.where(p, mid, hi))

        _, wr = lax.fori_loop(0, 6, rstep, (cb, jnp.int32(NCHUNK)))

        @pl.loop(wl, ca, init_carry=seed_ks)
        def left_ks(c, ks):
            return scan_chunk(c, ks)

        @pl.loop(cb, wr, init_carry=left_ks)
        def right_ks(c, ks):
            return scan_chunk(c, ks)

        m1, m2, m3, f1, f2, f3 = right_ks

        off = pl.ds(i * 16, 16)
        bd1[off] = m1
        bd2[off] = m2
        bd3[off] = m3
        bf1[off] = f1
        bf2[off] = f2
        bf3[off] = f3
        return carry_

    lax.fori_loop(0, QW // 16, qstep, 0)

    dst = pl.ds((b * 16 + s) * QW, QW)
    pltpu.sync_copy(bf1, o_f1.at[dst])
    pltpu.sync_copy(bf2, o_f2.at[dst])
    pltpu.sync_copy(bf3, o_f3.at[dst])
    pltpu.sync_copy(bd1, o_d1.at[dst])
    pltpu.sync_copy(bd2, o_d2.at[dst])
    pltpu.sync_copy(bd3, o_d3.at[dst])


def _sht_kernel(f1_ref, f2_ref, f3_ref, d1_ref, d2_ref, d3_ref,
                t_ref, cos_ref, wt_ref, rw_ref, out_ref):
    w1 = jnp.sqrt(jnp.maximum(d1_ref[...], 1e-12))
    w2 = jnp.sqrt(jnp.maximum(d2_ref[...], 1e-12))
    w3 = jnp.sqrt(jnp.maximum(d3_ref[...], 1e-12))
    x = (f1_ref[...] * w1 + f2_ref[...] * w2 + f3_ref[...] * w3) / (w1 + w2 + w3)
    xr = lax.dot(x, cos_ref[...], precision=lax.Precision.HIGHEST,
                 preferred_element_type=jnp.float32)
    xr = xr * np.float32(2.0 * _PI / NLON)
    wt = wt_ref[...]
    rw = rw_ref[...]
    loss = jnp.float32(0.0)
    for b in range(B):
        xb = xr[b * NLAT:(b + 1) * NLAT]
        cb = jnp.sum(wt * xb[:, None, :], axis=0)
        resid = cb - t_ref[b]
        loss = loss + jnp.sum(resid * resid * rw)
    out_ref[...] = (loss / B).reshape(1, 1)


def kernel(pred, target_coeffs):
    px = pred[:, :, 0].reshape(B, 1, N)
    py = pred[:, :, 1].reshape(B, 1, N)
    pz = pred[:, :, 2].reshape(B, 1, N)

    sxsp, sysp, ftsp = pl.pallas_call(
        _sph_sort_kernel,
        grid=(B,),
        in_specs=[pl.BlockSpec((1, 1, N), lambda b: (b, 0, 0))] * 3,
        out_specs=[pl.BlockSpec((1, N, 16), lambda b: (b, 0, 0))] * 3,
        out_shape=[jax.ShapeDtypeStruct((B, N, 16), jnp.float32)] * 3,
    )(px, py, pz)

    f1, f2, f3, d1, d2, d3 = _sc_knn(
        sxsp.reshape(B, N * 16), sysp.reshape(B, N * 16), ftsp.reshape(B, N * 16))

    shp = (B * NLAT, NLON)
    loss = pl.pallas_call(
        _sht_kernel,
        out_shape=jax.ShapeDtypeStruct((1, 1), jnp.float32),
    )(f1.reshape(shp), f2.reshape(shp), f3.reshape(shp),
      d1.reshape(shp), d2.reshape(shp), d3.reshape(shp),
      target_coeffs, jnp.asarray(_COS), jnp.asarray(_WTP), jnp.asarray(_RW))
    return loss[0, 0]
